# Initial kernel scaffold; baseline (speedup 1.0000x reference)
import functools

import jax
import jax.numpy as jnp
from jax import lax
from jax.experimental import pallas as pl
from jax.experimental.pallas import tpu as pltpu
from jax.experimental.pallas import tpu_sc as plsc

_NC, _NS, _L = 2, 16, 16
_NW = _NC * _NS
_B = 16384
_COLS = 3
_CATS = 4096
_RANK = 16
_BPW = _B // _NW
_CHUNK = 128
_GROUPS = _BPW // _L

_mesh = plsc.VectorSubcoreMesh(
    core_axis_name="c", subcore_axis_name="s",
    num_cores=_NC, num_subcores=_NS)


@functools.partial(
    pl.kernel,
    out_type=jax.ShapeDtypeStruct((_B,), jnp.float32),
    mesh=_mesh,
    scratch_types=[
        pltpu.VMEM((_COLS, _BPW), jnp.int32),
        pltpu.VMEM((_COLS, _BPW), jnp.int32),
        pltpu.VMEM((_COLS, _BPW, _RANK), jnp.float32),
        pltpu.VMEM((_COLS, _BPW, _RANK), jnp.float32),
        pltpu.VMEM((_COLS * _CATS,), jnp.float32),
        pltpu.VMEM((_BPW,), jnp.float32),
        pltpu.SemaphoreType.DMA,
    ],
)
def _index_kernel(sf_hbm, stds_hbm, xo_hbm, yo_hbm, out_hbm,
                  xi_v, yi_v, rx_v, ry_v, stds_v, acc_v, sem):
    wid = lax.axis_index("s") * _NC + lax.axis_index("c")
    base = wid * _BPW

    pltpu.sync_copy(stds_hbm, stds_v)
    for i in range(_COLS):
        pltpu.sync_copy(xo_hbm.at[i, pl.ds(base, _BPW)], xi_v.at[i])
        pltpu.sync_copy(yo_hbm.at[i, pl.ds(base, _BPW)], yi_v.at[i])

    copies = []
    for i in range(_COLS):
        for j in range(_BPW // _CHUNK):
            sl = pl.ds(j * _CHUNK, _CHUNK)
            copies.append(
                pltpu.async_copy(sf_hbm.at[xi_v.at[i, sl]], rx_v.at[i, sl], sem))
            copies.append(
                pltpu.async_copy(sf_hbm.at[yi_v.at[i, sl]], ry_v.at[i, sl], sem))
    for c in copies:
        c.wait()

    iota = lax.iota(jnp.int32, _L)

    def body(g, carry):
        b0 = g * _L
        acc = jnp.zeros((_L,), jnp.float32)
        rows = b0 + iota
        for i in range(_COLS):
            xv = xi_v[i, pl.ds(b0, _L)]
            yv = yi_v[i, pl.ds(b0, _L)]
            sv = plsc.load_gather(stds_v, [xv])
            acc = acc + jnp.where(xv == yv, sv * sv, jnp.zeros((_L,), jnp.float32))
            for r in range(_RANK):
                cols = jnp.full((_L,), r, jnp.int32)
                fx = plsc.load_gather(rx_v.at[i], [rows, cols])
                fy = plsc.load_gather(ry_v.at[i], [rows, cols])
                p = fx * fy
                acc = acc + p * p
        acc_v[pl.ds(b0, _L)] = acc
        return carry

    lax.fori_loop(0, _GROUPS, body, 0)
    pltpu.sync_copy(acc_v, out_hbm.at[pl.ds(base, _BPW)])


def kernel(x, y, sqrt_covar_factors, stds):
    off = jnp.arange(_COLS, dtype=jnp.int32) * _CATS
    xo = (x + off[None, :]).T
    yo = (y + off[None, :]).T
    sf_flat = sqrt_covar_factors.reshape(_COLS * _CATS, _RANK)
    stds_flat = stds.reshape(_COLS * _CATS)
    return _index_kernel(sf_flat, stds_flat, xo, yo)


# speedup vs baseline: 2.1808x; 2.1808x over previous
import functools

import jax
import jax.numpy as jnp
from jax import lax
from jax.experimental import pallas as pl
from jax.experimental.pallas import tpu as pltpu
from jax.experimental.pallas import tpu_sc as plsc

_NC, _NS, _L = 2, 16, 16
_NW = _NC * _NS
_B = 16384
_COLS = 3
_CATS = 4096
_RANK = 16
_BPW = _B // _NW
_CHUNK = 128
_GROUPS = _BPW // _L

_mesh = plsc.VectorSubcoreMesh(
    core_axis_name="c", subcore_axis_name="s",
    num_cores=_NC, num_subcores=_NS)


@functools.partial(
    pl.kernel,
    out_type=jax.ShapeDtypeStruct((_B,), jnp.float32),
    mesh=_mesh,
    compiler_params=pltpu.CompilerParams(
        needs_layout_passes=False, use_tc_tiling_on_sc=False),
    scratch_types=[
        pltpu.VMEM((_COLS * _BPW,), jnp.int32),
        pltpu.VMEM((_COLS * _BPW,), jnp.int32),
        pltpu.VMEM((_COLS * _BPW, _RANK), jnp.float32),
        pltpu.VMEM((_COLS * _BPW, _RANK), jnp.float32),
        pltpu.VMEM((_COLS * _CATS,), jnp.float32),
        pltpu.VMEM((_BPW,), jnp.float32),
        pltpu.SemaphoreType.DMA,
    ],
)
def _index_kernel(sf_hbm, stds_hbm, xo_hbm, yo_hbm, out_hbm,
                  xi_v, yi_v, rx_v, ry_v, stds_v, acc_v, sem):
    wid = lax.axis_index("s") * _NC + lax.axis_index("c")
    base = wid * _BPW

    pltpu.sync_copy(stds_hbm, stds_v)
    for i in range(_COLS):
        pltpu.sync_copy(xo_hbm.at[pl.ds(i * _B + base, _BPW)],
                        xi_v.at[pl.ds(i * _BPW, _BPW)])
        pltpu.sync_copy(yo_hbm.at[pl.ds(i * _B + base, _BPW)],
                        yi_v.at[pl.ds(i * _BPW, _BPW)])

    copies = []
    for i in range(_COLS):
        for j in range(_BPW // _CHUNK):
            lo = i * _BPW + j * _CHUNK
            sl = pl.ds(lo, _CHUNK)
            copies.append(
                pltpu.async_copy(sf_hbm.at[xi_v.at[sl]], rx_v.at[sl], sem))
            copies.append(
                pltpu.async_copy(sf_hbm.at[yi_v.at[sl]], ry_v.at[sl], sem))
    for c in copies:
        c.wait()

    iota = lax.iota(jnp.int32, _L)

    def body(g, carry):
        b0 = g * _L
        acc = jnp.zeros((_L,), jnp.float32)
        for i in range(_COLS):
            xv = xi_v[pl.ds(i * _BPW + b0, _L)]
            yv = yi_v[pl.ds(i * _BPW + b0, _L)]
            sv = plsc.load_gather(stds_v, [xv])
            acc = acc + jnp.where(xv == yv, sv * sv, jnp.zeros((_L,), jnp.float32))
            rows = i * _BPW + b0 + iota
            for r in range(_RANK):
                cols = jnp.full((_L,), r, jnp.int32)
                fx = plsc.load_gather(rx_v, [rows, cols])
                fy = plsc.load_gather(ry_v, [rows, cols])
                p = fx * fy
                acc = acc + p * p
        acc_v[pl.ds(b0, _L)] = acc
        return carry

    lax.fori_loop(0, _GROUPS, body, 0)
    pltpu.sync_copy(acc_v, out_hbm.at[pl.ds(base, _BPW)])


def kernel(x, y, sqrt_covar_factors, stds):
    off = jnp.arange(_COLS, dtype=jnp.int32) * _CATS
    xo = (x + off[None, :]).T.reshape(_COLS * _B)
    yo = (y + off[None, :]).T.reshape(_COLS * _B)
    sf_flat = sqrt_covar_factors.reshape(_COLS * _CATS, _RANK)
    stds_flat = stds.reshape(_COLS * _CATS)
    return _index_kernel(sf_flat, stds_flat, xo, yo)
